# Initial kernel scaffold; baseline (speedup 1.0000x reference)
import jax
import jax.numpy as jnp
from jax.experimental import pallas as pl


def _probe_body(in_ref, o_ref):
    x = in_ref[...]
    y = jnp.reshape(x, (x.shape[0] * 8, 16))
    a = y[:, 0:8]
    b = y[:, 8:16]
    z = a - b
    w = jnp.concatenate([z, z], axis=1)
    o_ref[...] = jnp.reshape(w, (x.shape[0], 128))


def kernel(x, edge_index, g1, mu1, sigma1, root1, b1, g2, mu2, sigma2, root2, b2):
    E = edge_index.shape[1]
    BR = 1000
    xsd = jnp.zeros((E * 16,), jnp.float32).reshape(E // 8, 128)
    out = pl.pallas_call(
        _probe_body,
        grid=(xsd.shape[0] // BR,),
        in_specs=[pl.BlockSpec((BR, 128), lambda i: (i, 0))],
        out_specs=pl.BlockSpec((BR, 128), lambda i: (i, 0)),
        out_shape=jax.ShapeDtypeStruct((E // 8, 128), jnp.float32),
    )(xsd)
    s = out.reshape(E, 16)[:, :1].sum()
    n = x.shape[0]
    return jnp.zeros((n, 8), jnp.float32) + s


# speedup vs baseline: 6.3898x; 6.3898x over previous
import functools

import jax
import jax.numpy as jnp
from jax import lax
from jax.experimental import pallas as pl
from jax.experimental.pallas import tpu as pltpu
from jax.experimental.pallas import tpu_sc as plsc

_EPS = 1e-15
_NC = 2
_NS = 16
_NW = _NC * _NS
_GRP = 128


def _mesh():
    return plsc.VectorSubcoreMesh(core_axis_name="c", subcore_axis_name="s")


_SC_PARAMS = pltpu.CompilerParams(use_tc_tiling_on_sc=False)


def _wid():
    return lax.axis_index("s") * _NC + lax.axis_index("c")


def _gather_body(nu, tail, rpu, tab_hbm, idxr_hbm, out_hbm, irow, bs, sem1):
    wid = _wid()
    nu_w = (nu + _NW - 1 - wid) // _NW

    def unit(r0, sup):
        pltpu.sync_copy(idxr_hbm.at[pl.ds(r0, sup)], irow.at[pl.ds(0, sup)])
        cps = []
        for j in range(sup):
            cps.append(pltpu.async_copy(
                tab_hbm.at[irow.at[j]], bs.at[pl.ds(j * _GRP, _GRP)], sem1))
        for cp in cps:
            cp.wait()
        pltpu.sync_copy(bs.at[pl.ds(0, sup * _GRP)],
                        out_hbm.at[pl.ds(r0 * _GRP, sup * _GRP)])

    def it_body(j, carry):
        unit((wid + _NW * j) * rpu, rpu)
        return carry

    lax.fori_loop(0, nu_w, it_body, 0)
    if tail:
        @pl.when(wid == 0)
        def _():
            unit(nu * rpu, tail)


def _tile_rows(n):
    big = -(-n // _NS)
    big = -(-big // 8) * 8
    last = n - big * (_NS - 1)
    assert last > 0
    return big, last


def _scatter_body(nu, tail, with_cnt, msg_hbm, colr_hbm, zz_hbm, *args):
    if with_cnt:
        (zc_hbm, ones_hbm, agg_hbm, cnt_hbm,
         icol, mbuf, ones_v, acc, accc, sem1, sem2) = args
    else:
        (agg_hbm, icol, mbuf, acc, sem1) = args
    c = lax.axis_index("c")
    s = lax.axis_index("s")
    wid = s * _NC + c
    n = acc.shape[0]
    big, last = _tile_rows(n)
    r0 = s * big

    def init(nrow):
        pltpu.sync_copy(zz_hbm.at[pl.ds(r0, nrow)], acc.at[pl.ds(r0, nrow)])
        if with_cnt:
            pltpu.sync_copy(zc_hbm.at[pl.ds(r0, nrow)],
                            accc.at[pl.ds(r0, nrow)])

    @pl.when(s < _NS - 1)
    def _():
        init(big)

    @pl.when(s == _NS - 1)
    def _():
        init(last)

    if with_cnt:
        pltpu.sync_copy(ones_hbm, ones_v)
    plsc.subcore_barrier()

    def unit(g0, sup):
        pltpu.sync_copy(colr_hbm.at[pl.ds(g0, sup)], icol.at[pl.ds(0, sup)])
        pltpu.sync_copy(msg_hbm.at[pl.ds(g0 * _GRP, sup * _GRP)],
                        mbuf.at[pl.ds(0, sup * _GRP)])
        cps = []
        for j in range(sup):
            cps.append(pltpu.async_copy(
                mbuf.at[pl.ds(j * _GRP, _GRP)], acc.at[icol.at[j]],
                sem1, add=True))
            if with_cnt:
                cps.append(pltpu.async_copy(
                    ones_v, accc.at[icol.at[j]], sem2, add=True))
        for cp in cps:
            cp.wait()

    def it_body(j, carry):
        unit((wid + _NW * j) * 8, 8)
        return carry

    nu_w = (nu + _NW - 1 - wid) // _NW
    lax.fori_loop(0, nu_w, it_body, 0)
    if tail:
        @pl.when(wid == 0)
        def _():
            unit(nu * 8, tail)

    plsc.subcore_barrier()

    def fini(nrow):
        pltpu.sync_copy(acc.at[pl.ds(r0, nrow)],
                        agg_hbm.at[c, pl.ds(r0, nrow)])
        if with_cnt:
            pltpu.sync_copy(accc.at[pl.ds(r0, nrow)],
                            cnt_hbm.at[c, pl.ds(r0, nrow)])

    @pl.when(s < _NS - 1)
    def _():
        fini(big)

    @pl.when(s == _NS - 1)
    def _():
        fini(last)


def _lane_iota(shape):
    return lax.broadcasted_iota(jnp.int32, shape, len(shape) - 1)


def _edge1_body(nk, xsd_ref, m1_ref, w1_ref, mn1_ref, m2_ref, w2_ref,
                mn2_ref, g1k_ref, msg_ref, gs2_ref):
    f32 = jnp.float32
    x = xsd_ref[...]
    l = _lane_iota(x.shape)
    y = jnp.take_along_axis(x, l ^ 8, axis=1)
    ps = x - y
    sq = ps * ps
    e1 = (jnp.dot(ps, m1_ref[...], preferred_element_type=f32)
          - jnp.dot(sq, w1_ref[...], preferred_element_type=f32)
          - mn1_ref[...])
    g1 = jnp.exp(e1)
    e2 = (jnp.dot(ps, m2_ref[...], preferred_element_type=f32)
          - jnp.dot(sq, w2_ref[...], preferred_element_type=f32)
          - mn2_ref[...])
    gs2_ref[...] = jnp.exp(e2)
    base = (l >> 4) << 4
    msg = jnp.zeros(x.shape, f32)
    for k in range(nk):
        bk = jnp.take_along_axis(g1, base + k, axis=1)
        msg = msg + bk * jnp.dot(x, g1k_ref[k], preferred_element_type=f32)
    msg_ref[...] = msg


def _edge2_body(nk, hs_ref, gs2_ref, g2k_ref, msg2_ref):
    f32 = jnp.float32
    hs = hs_ref[...]
    g2 = gs2_ref[...]
    l = _lane_iota(hs.shape)
    base = (l >> 4) << 4
    msg = jnp.zeros(hs.shape, f32)
    for k in range(nk):
        bk = jnp.take_along_axis(g2, base + k, axis=1)
        msg = msg + bk * jnp.dot(hs, g2k_ref[k], preferred_element_type=f32)
    msg2_ref[...] = msg


def _node1_body(agg_ref, cnt_ref, x_ref, r1t_ref, b1_ref, h_ref, invc_ref):
    f32 = jnp.float32
    agg = agg_ref[...]
    cnt = cnt_ref[...]
    asum = agg[0] + agg[1]
    csum = cnt[0] + cnt[1]
    ic = 1.0 / jnp.maximum(csum[:, 0:1], 1.0)
    a = (asum * ic + jnp.dot(x_ref[...], r1t_ref[...],
                             preferred_element_type=f32) + b1_ref[...])
    h_ref[...] = jnp.where(a > 0.0, a, jnp.exp(jnp.minimum(a, 0.0)) - 1.0)
    invc_ref[...] = ic


def _node2_body(agg2_ref, invc_ref, h_ref, r2t_ref, b2_ref, out_ref):
    f32 = jnp.float32
    a = agg2_ref[...]
    dout = out_ref.shape[1]
    out_ref[...] = ((a[0] + a[1])[:, 0:dout] * invc_ref[...]
                    + jnp.dot(h_ref[...], r2t_ref[...],
                              preferred_element_type=f32)
                    + b2_ref[...])


def _blockdiag(block):
    return jnp.kron(jnp.eye(8, dtype=block.dtype), block)


def kernel(x, edge_index, g1, mu1, sigma1, root1, b1, g2, mu2, sigma2, root2,
           b2):
    f32 = jnp.float32
    n, din = x.shape
    e = edge_index.shape[1]
    k, dim = mu1.shape
    hid = root1.shape[0]
    dout = root2.shape[0]
    ngrp = e // _GRP
    assert e % _GRP == 0

    row = edge_index[0]
    col = edge_index[1]
    idx_int = edge_index.T.reshape(2 * ngrp, _GRP)
    rowr = row.reshape(ngrp, _GRP)
    colr = col.reshape(ngrp, _GRP)

    nu2 = 2 * ngrp // 16
    tail2 = 2 * ngrp % 16
    gathx = pl.kernel(
        functools.partial(_gather_body, nu2, tail2, 16),
        out_type=jax.ShapeDtypeStruct((2 * e, din), f32),
        mesh=_mesh(),
        compiler_params=_SC_PARAMS,
        scratch_types=(
            pltpu.VMEM((16, _GRP), jnp.int32),
            pltpu.VMEM((16 * _GRP, din), f32),
            pltpu.SemaphoreType.DMA,
        ),
    )
    xsd = gathx(x, idx_int).reshape(e // 8, 128)

    def gauss_mats(mu, sigma):
        w = 1.0 / (_EPS + sigma * sigma)
        mblk = jnp.zeros((16, 16), f32).at[:dim, :k].set((mu * w).T)
        wblk = jnp.zeros((16, 16), f32).at[:dim, :k].set((0.5 * w).T)
        mn = (0.5 * (mu * mu * w)).sum(axis=1)
        mnrow = jnp.concatenate([mn, jnp.full((16 - k,), 1e30, f32)])
        return (_blockdiag(mblk), _blockdiag(wblk),
                jnp.tile(mnrow, (8,))[None, :])

    m1m, w1m, mn1r = gauss_mats(mu1, sigma1)
    m2m, w2m, mn2r = gauss_mats(mu2, sigma2)
    g1r = g1.reshape(din, k, hid)
    g1k = jnp.stack([
        _blockdiag(jnp.zeros((16, 16), f32).at[:din, :hid].set(g1r[:, kk, :]))
        for kk in range(k)])
    g2r = g2.reshape(hid, k, dout)
    g2k = jnp.stack([
        _blockdiag(jnp.zeros((16, 16), f32).at[:hid, :dout].set(g2r[:, kk, :]))
        for kk in range(k)])

    br = 2000
    nrow8 = e // 8
    full = lambda *s: pl.BlockSpec(s, lambda i: tuple(0 for _ in s))
    blk = lambda r: pl.BlockSpec((r, 128), lambda i: (i, 0))
    msg1, gs2 = pl.pallas_call(
        functools.partial(_edge1_body, k),
        grid=(nrow8 // br,),
        in_specs=[blk(br), full(128, 128), full(128, 128), full(1, 128),
                  full(128, 128), full(128, 128), full(1, 128),
                  full(k, 128, 128)],
        out_specs=[blk(br), blk(br)],
        out_shape=[jax.ShapeDtypeStruct((nrow8, 128), f32),
                   jax.ShapeDtypeStruct((nrow8, 128), f32)],
    )(xsd, m1m, w1m, mn1r, m2m, w2m, mn2r, g1k)

    nu = ngrp // 8
    tail = ngrp % 8
    z16 = jnp.zeros((n, 16), f32)
    z8 = jnp.zeros((n, k), f32)
    ones8 = jnp.ones((_GRP, k), f32)
    scat1 = pl.kernel(
        functools.partial(_scatter_body, nu, tail, True),
        out_type=(jax.ShapeDtypeStruct((_NC, n, 16), f32),
                  jax.ShapeDtypeStruct((_NC, n, k), f32)),
        mesh=_mesh(),
        compiler_params=_SC_PARAMS,
        scratch_types=(
            pltpu.VMEM((8, _GRP), jnp.int32),
            pltpu.VMEM((8 * _GRP, 16), f32),
            pltpu.VMEM((_GRP, k), f32),
            pltpu.VMEM_SHARED((n, 16), f32),
            pltpu.VMEM_SHARED((n, k), f32),
            pltpu.SemaphoreType.DMA,
            pltpu.SemaphoreType.DMA,
        ),
    )
    agg1, cnt = scat1(msg1.reshape(e, 16), colr, z16, z8, ones8)

    bn = 5000
    h, invc = pl.pallas_call(
        _node1_body,
        grid=(n // bn,),
        in_specs=[
            pl.BlockSpec((_NC, bn, 16), lambda i: (0, i, 0)),
            pl.BlockSpec((_NC, bn, k), lambda i: (0, i, 0)),
            pl.BlockSpec((bn, din), lambda i: (i, 0)),
            full(din, hid), full(1, hid),
        ],
        out_specs=[pl.BlockSpec((bn, hid), lambda i: (i, 0)),
                   pl.BlockSpec((bn, 1), lambda i: (i, 0))],
        out_shape=[jax.ShapeDtypeStruct((n, hid), f32),
                   jax.ShapeDtypeStruct((n, 1), f32)],
    )(agg1, cnt, x, root1.T.astype(f32), b1[None, :].astype(f32))

    gathh = pl.kernel(
        functools.partial(_gather_body, nu, tail, 8),
        out_type=jax.ShapeDtypeStruct((e, hid), f32),
        mesh=_mesh(),
        compiler_params=_SC_PARAMS,
        scratch_types=(
            pltpu.VMEM((8, _GRP), jnp.int32),
            pltpu.VMEM((8 * _GRP, hid), f32),
            pltpu.SemaphoreType.DMA,
        ),
    )
    hs = gathh(h, rowr).reshape(e // 8, 128)

    msg2 = pl.pallas_call(
        functools.partial(_edge2_body, k),
        grid=(nrow8 // br,),
        in_specs=[blk(br), blk(br), full(k, 128, 128)],
        out_specs=blk(br),
        out_shape=jax.ShapeDtypeStruct((nrow8, 128), f32),
    )(hs, gs2, g2k)

    scat2 = pl.kernel(
        functools.partial(_scatter_body, nu, tail, False),
        out_type=jax.ShapeDtypeStruct((_NC, n, 16), f32),
        mesh=_mesh(),
        compiler_params=_SC_PARAMS,
        scratch_types=(
            pltpu.VMEM((8, _GRP), jnp.int32),
            pltpu.VMEM((8 * _GRP, 16), f32),
            pltpu.VMEM_SHARED((n, 16), f32),
            pltpu.SemaphoreType.DMA,
        ),
    )
    agg2 = scat2(msg2.reshape(e, 16), colr, z16)

    out = pl.pallas_call(
        _node2_body,
        grid=(n // bn,),
        in_specs=[
            pl.BlockSpec((_NC, bn, 16), lambda i: (0, i, 0)),
            pl.BlockSpec((bn, 1), lambda i: (i, 0)),
            pl.BlockSpec((bn, hid), lambda i: (i, 0)),
            full(hid, dout), full(1, dout),
        ],
        out_specs=pl.BlockSpec((bn, dout), lambda i: (i, 0)),
        out_shape=jax.ShapeDtypeStruct((n, dout), f32),
    )(agg2, invc, h, root2.T.astype(f32), b2[None, :].astype(f32))
    return out


# speedup vs baseline: 12.0418x; 1.8845x over previous
import functools

import jax
import jax.numpy as jnp
from jax import lax
from jax.experimental import pallas as pl
from jax.experimental.pallas import tpu as pltpu
from jax.experimental.pallas import tpu_sc as plsc

_EPS = 1e-15
_NC = 2
_NS = 16
_NW = _NC * _NS
_GRP = 128


def _mesh():
    return plsc.VectorSubcoreMesh(core_axis_name="c", subcore_axis_name="s")


_SC_PARAMS = pltpu.CompilerParams(use_tc_tiling_on_sc=False)


def _wid():
    return lax.axis_index("s") * _NC + lax.axis_index("c")


def _gather2_body(nu, tail, x_hbm, rowr_hbm, colr_hbm, xs_hbm, xd_hbm,
                  irow, icol, bs, bd, sem1, sem2):
    wid = _wid()
    nu_w = (nu + _NW - 1 - wid) // _NW

    def unit(g0, sup):
        pltpu.sync_copy(rowr_hbm.at[pl.ds(g0, sup)], irow.at[pl.ds(0, sup)])
        pltpu.sync_copy(colr_hbm.at[pl.ds(g0, sup)], icol.at[pl.ds(0, sup)])
        cps = []
        for j in range(sup):
            cps.append(pltpu.async_copy(
                x_hbm.at[irow.at[j]], bs.at[pl.ds(j * _GRP, _GRP)], sem1))
            cps.append(pltpu.async_copy(
                x_hbm.at[icol.at[j]], bd.at[pl.ds(j * _GRP, _GRP)], sem2))
        for cp in cps:
            cp.wait()
        e0 = g0 * _GRP
        pltpu.sync_copy(bs.at[pl.ds(0, sup * _GRP)],
                        xs_hbm.at[pl.ds(e0, sup * _GRP)])
        pltpu.sync_copy(bd.at[pl.ds(0, sup * _GRP)],
                        xd_hbm.at[pl.ds(e0, sup * _GRP)])

    def it_body(j, carry):
        unit((wid + _NW * j) * 8, 8)
        return carry

    lax.fori_loop(0, nu_w, it_body, 0)
    if tail:
        @pl.when(wid == 0)
        def _():
            unit(nu * 8, tail)


def _gather1_body(nu, tail, tab_hbm, rowr_hbm, out_hbm, irow, bs, sem1):
    wid = _wid()
    nu_w = (nu + _NW - 1 - wid) // _NW

    def unit(g0, sup):
        pltpu.sync_copy(rowr_hbm.at[pl.ds(g0, sup)], irow.at[pl.ds(0, sup)])
        cps = []
        for j in range(sup):
            cps.append(pltpu.async_copy(
                tab_hbm.at[irow.at[j]], bs.at[pl.ds(j * _GRP, _GRP)], sem1))
        for cp in cps:
            cp.wait()
        pltpu.sync_copy(bs.at[pl.ds(0, sup * _GRP)],
                        out_hbm.at[pl.ds(g0 * _GRP, sup * _GRP)])

    def it_body(j, carry):
        unit((wid + _NW * j) * 8, 8)
        return carry

    lax.fori_loop(0, nu_w, it_body, 0)
    if tail:
        @pl.when(wid == 0)
        def _():
            unit(nu * 8, tail)


def _tile_rows(n):
    big = -(-n // _NS)
    big = -(-big // 8) * 8
    last = n - big * (_NS - 1)
    assert last > 0
    return big, last


def _scatter_body(nu, tail, with_cnt, msg_hbm, colr_hbm, zz_hbm, *args):
    if with_cnt:
        (zc_hbm, ones_hbm, agg_hbm, cnt_hbm,
         icol, mbuf, ones_v, acc, accc, sem1, sem2) = args
    else:
        (agg_hbm, icol, mbuf, acc, sem1) = args
    c = lax.axis_index("c")
    s = lax.axis_index("s")
    wid = s * _NC + c
    n = acc.shape[0]
    big, last = _tile_rows(n)
    r0 = s * big

    def init(nrow):
        pltpu.sync_copy(zz_hbm.at[pl.ds(r0, nrow)], acc.at[pl.ds(r0, nrow)])
        if with_cnt:
            pltpu.sync_copy(zc_hbm.at[pl.ds(r0, nrow)],
                            accc.at[pl.ds(r0, nrow)])

    @pl.when(s < _NS - 1)
    def _():
        init(big)

    @pl.when(s == _NS - 1)
    def _():
        init(last)

    if with_cnt:
        pltpu.sync_copy(ones_hbm, ones_v)
    plsc.subcore_barrier()

    def unit(g0, sup):
        pltpu.sync_copy(colr_hbm.at[pl.ds(g0, sup)], icol.at[pl.ds(0, sup)])
        pltpu.sync_copy(msg_hbm.at[pl.ds(g0 * _GRP, sup * _GRP)],
                        mbuf.at[pl.ds(0, sup * _GRP)])
        cps = []
        for j in range(sup):
            cps.append(pltpu.async_copy(
                mbuf.at[pl.ds(j * _GRP, _GRP)], acc.at[icol.at[j]],
                sem1, add=True))
            if with_cnt:
                cps.append(pltpu.async_copy(
                    ones_v, accc.at[icol.at[j]], sem2, add=True))
        for cp in cps:
            cp.wait()

    def it_body(j, carry):
        unit((wid + _NW * j) * 8, 8)
        return carry

    nu_w = (nu + _NW - 1 - wid) // _NW
    lax.fori_loop(0, nu_w, it_body, 0)
    if tail:
        @pl.when(wid == 0)
        def _():
            unit(nu * 8, tail)

    plsc.subcore_barrier()

    def fini(nrow):
        pltpu.sync_copy(acc.at[pl.ds(r0, nrow)],
                        agg_hbm.at[c, pl.ds(r0, nrow)])
        if with_cnt:
            pltpu.sync_copy(accc.at[pl.ds(r0, nrow)],
                            cnt_hbm.at[c, pl.ds(r0, nrow)])

    @pl.when(s < _NS - 1)
    def _():
        fini(big)

    @pl.when(s == _NS - 1)
    def _():
        fini(last)


def _edge1_body(nk, xs_ref, xd_ref, m1_ref, w1_ref, mn1_ref, m2_ref, w2_ref,
                mn2_ref, g1k_ref, msg_ref, gs2_ref):
    f32 = jnp.float32
    xs = xs_ref[...]
    ps = xs - xd_ref[...]
    sq = ps * ps
    e1 = (jnp.dot(ps, m1_ref[...], preferred_element_type=f32)
          - jnp.dot(sq, w1_ref[...], preferred_element_type=f32)
          - mn1_ref[...])
    g1 = jnp.exp(e1)
    e2 = (jnp.dot(ps, m2_ref[...], preferred_element_type=f32)
          - jnp.dot(sq, w2_ref[...], preferred_element_type=f32)
          - mn2_ref[...])
    gs2_ref[...] = jnp.exp(e2)
    l = lax.broadcasted_iota(jnp.int32, (xs.shape[0], 256), 1)
    base = (l >> 4) << 3
    msg = jnp.zeros((xs.shape[0], 256), f32)
    for k in range(nk):
        bk = jnp.take_along_axis(g1, base + k, axis=1)
        msg = msg + bk * jnp.dot(xs, g1k_ref[k], preferred_element_type=f32)
    msg_ref[...] = msg


def _edge2_body(nk, hs_ref, gs2_ref, g2k_ref, msg2_ref):
    f32 = jnp.float32
    hs = hs_ref[...]
    g2 = gs2_ref[...]
    l = lax.broadcasted_iota(jnp.int32, g2.shape, 1)
    base = (l >> 3) << 3
    msg = jnp.zeros(g2.shape, f32)
    for k in range(nk):
        bk = jnp.take_along_axis(g2, base + k, axis=1)
        msg = msg + bk * jnp.dot(hs, g2k_ref[k], preferred_element_type=f32)
    msg2_ref[...] = msg


def _node1_body(agg_ref, cnt_ref, x_ref, r1t_ref, b1_ref, h_ref, invc_ref):
    f32 = jnp.float32
    agg = agg_ref[...]
    cnt = cnt_ref[...]
    asum = agg[0] + agg[1]
    csum = cnt[0] + cnt[1]
    ic = 1.0 / jnp.maximum(csum[:, 0:1], 1.0)
    a = (asum * ic + jnp.dot(x_ref[...], r1t_ref[...],
                             preferred_element_type=f32) + b1_ref[...])
    h_ref[...] = jnp.where(a > 0.0, a, jnp.exp(jnp.minimum(a, 0.0)) - 1.0)
    invc_ref[...] = ic


def _node2_body(agg2_ref, invc_ref, h_ref, r2t_ref, b2_ref, out_ref):
    f32 = jnp.float32
    a = agg2_ref[...]
    out_ref[...] = ((a[0] + a[1]) * invc_ref[...]
                    + jnp.dot(h_ref[...], r2t_ref[...],
                              preferred_element_type=f32)
                    + b2_ref[...])


def _blockdiag16(block):
    return jnp.kron(jnp.eye(16, dtype=block.dtype), block)


def kernel(x, edge_index, g1, mu1, sigma1, root1, b1, g2, mu2, sigma2, root2,
           b2):
    f32 = jnp.float32
    n, din = x.shape
    e = edge_index.shape[1]
    k, dim = mu1.shape
    hid = root1.shape[0]
    dout = root2.shape[0]
    ngrp = e // _GRP
    assert e % _GRP == 0

    row = edge_index[0]
    col = edge_index[1]
    rowr = row.reshape(ngrp, _GRP)
    colr = col.reshape(ngrp, _GRP)
    nu = ngrp // 8
    tail = ngrp % 8

    gathx = pl.kernel(
        functools.partial(_gather2_body, nu, tail),
        out_type=(jax.ShapeDtypeStruct((e, din), f32),
                  jax.ShapeDtypeStruct((e, din), f32)),
        mesh=_mesh(),
        compiler_params=_SC_PARAMS,
        scratch_types=(
            pltpu.VMEM((8, _GRP), jnp.int32),
            pltpu.VMEM((8, _GRP), jnp.int32),
            pltpu.VMEM((8 * _GRP, din), f32),
            pltpu.VMEM((8 * _GRP, din), f32),
            pltpu.SemaphoreType.DMA,
            pltpu.SemaphoreType.DMA,
        ),
    )
    xs, xd = gathx(x, rowr, colr)
    nrow16 = e // 16
    xs = xs.reshape(nrow16, 128)
    xd = xd.reshape(nrow16, 128)

    def gauss_mats(mu, sigma):
        w = 1.0 / (_EPS + sigma * sigma)
        mblk = jnp.zeros((8, 8), f32).at[:dim, :k].set((mu * w).T)
        wblk = jnp.zeros((8, 8), f32).at[:dim, :k].set((0.5 * w).T)
        mn = (0.5 * (mu * mu * w)).sum(axis=1)
        return (_blockdiag16(mblk), _blockdiag16(wblk),
                jnp.tile(mn, (16,))[None, :])

    m1m, w1m, mn1r = gauss_mats(mu1, sigma1)
    m2m, w2m, mn2r = gauss_mats(mu2, sigma2)
    g1r = g1.reshape(din, k, hid)
    g1k = jnp.stack([_blockdiag16(g1r[:, kk, :]) for kk in range(k)])
    g2r = g2.reshape(hid, k, dout)
    g2k = jnp.stack([_blockdiag16(g2r[:, kk, :]) for kk in range(k)])

    br = 2000
    full = lambda *s: pl.BlockSpec(s, lambda i: tuple(0 for _ in s))
    blk = lambda r, w: pl.BlockSpec((r, w), lambda i: (i, 0))
    msg1, gs2 = pl.pallas_call(
        functools.partial(_edge1_body, k),
        grid=(nrow16 // br,),
        in_specs=[blk(br, 128), blk(br, 128),
                  full(128, 128), full(128, 128), full(1, 128),
                  full(128, 128), full(128, 128), full(1, 128),
                  full(k, 128, 16 * hid)],
        out_specs=[blk(br, 16 * hid), blk(br, 128)],
        out_shape=[jax.ShapeDtypeStruct((nrow16, 16 * hid), f32),
                   jax.ShapeDtypeStruct((nrow16, 128), f32)],
    )(xs, xd, m1m, w1m, mn1r, m2m, w2m, mn2r, g1k)

    z16 = jnp.zeros((n, hid), f32)
    z8 = jnp.zeros((n, k), f32)
    ones8 = jnp.ones((_GRP, k), f32)
    scat1 = pl.kernel(
        functools.partial(_scatter_body, nu, tail, True),
        out_type=(jax.ShapeDtypeStruct((_NC, n, hid), f32),
                  jax.ShapeDtypeStruct((_NC, n, k), f32)),
        mesh=_mesh(),
        compiler_params=_SC_PARAMS,
        scratch_types=(
            pltpu.VMEM((8, _GRP), jnp.int32),
            pltpu.VMEM((8 * _GRP, hid), f32),
            pltpu.VMEM((_GRP, k), f32),
            pltpu.VMEM_SHARED((n, hid), f32),
            pltpu.VMEM_SHARED((n, k), f32),
            pltpu.SemaphoreType.DMA,
            pltpu.SemaphoreType.DMA,
        ),
    )
    agg1, cnt = scat1(msg1.reshape(e, hid), colr, z16, z8, ones8)

    bn = 5000
    h, invc = pl.pallas_call(
        _node1_body,
        grid=(n // bn,),
        in_specs=[
            pl.BlockSpec((_NC, bn, hid), lambda i: (0, i, 0)),
            pl.BlockSpec((_NC, bn, k), lambda i: (0, i, 0)),
            pl.BlockSpec((bn, din), lambda i: (i, 0)),
            full(din, hid), full(1, hid),
        ],
        out_specs=[pl.BlockSpec((bn, hid), lambda i: (i, 0)),
                   pl.BlockSpec((bn, 1), lambda i: (i, 0))],
        out_shape=[jax.ShapeDtypeStruct((n, hid), f32),
                   jax.ShapeDtypeStruct((n, 1), f32)],
    )(agg1, cnt, x, root1.T.astype(f32), b1[None, :].astype(f32))

    gathh = pl.kernel(
        functools.partial(_gather1_body, nu, tail),
        out_type=jax.ShapeDtypeStruct((e, hid), f32),
        mesh=_mesh(),
        compiler_params=_SC_PARAMS,
        scratch_types=(
            pltpu.VMEM((8, _GRP), jnp.int32),
            pltpu.VMEM((8 * _GRP, hid), f32),
            pltpu.SemaphoreType.DMA,
        ),
    )
    hs = gathh(h, rowr).reshape(nrow16, 16 * hid)

    msg2 = pl.pallas_call(
        functools.partial(_edge2_body, k),
        grid=(nrow16 // br,),
        in_specs=[blk(br, 16 * hid), blk(br, 128), full(k, 16 * hid, 128)],
        out_specs=blk(br, 128),
        out_shape=jax.ShapeDtypeStruct((nrow16, 128), f32),
    )(hs, gs2, g2k)

    zo = jnp.zeros((n, dout), f32)
    scat2 = pl.kernel(
        functools.partial(_scatter_body, nu, tail, False),
        out_type=jax.ShapeDtypeStruct((_NC, n, dout), f32),
        mesh=_mesh(),
        compiler_params=_SC_PARAMS,
        scratch_types=(
            pltpu.VMEM((8, _GRP), jnp.int32),
            pltpu.VMEM((8 * _GRP, dout), f32),
            pltpu.VMEM_SHARED((n, dout), f32),
            pltpu.SemaphoreType.DMA,
        ),
    )
    agg2 = scat2(msg2.reshape(e, dout), colr, zo)

    out = pl.pallas_call(
        _node2_body,
        grid=(n // bn,),
        in_specs=[
            pl.BlockSpec((_NC, bn, dout), lambda i: (0, i, 0)),
            pl.BlockSpec((bn, 1), lambda i: (i, 0)),
            pl.BlockSpec((bn, hid), lambda i: (i, 0)),
            full(hid, dout), full(1, dout),
        ],
        out_specs=pl.BlockSpec((bn, dout), lambda i: (i, 0)),
        out_shape=jax.ShapeDtypeStruct((n, dout), f32),
    )(agg2, invc, h, root2.T.astype(f32), b2[None, :].astype(f32))
    return out


# speedup vs baseline: 12.9414x; 1.0747x over previous
import functools

import jax
import jax.numpy as jnp
from jax import lax
from jax.experimental import pallas as pl
from jax.experimental.pallas import tpu as pltpu
from jax.experimental.pallas import tpu_sc as plsc

_EPS = 1e-15
_NC = 2
_NS = 16
_NW = _NC * _NS
_GRP = 128


def _mesh():
    return plsc.VectorSubcoreMesh(core_axis_name="c", subcore_axis_name="s")


_SC_PARAMS = pltpu.CompilerParams(use_tc_tiling_on_sc=False)


def _wid():
    return lax.axis_index("s") * _NC + lax.axis_index("c")


def _gather2_body(nu, tail, x_hbm, rowr_hbm, colr_hbm, xs_hbm, xd_hbm,
                  irow, icol, bs, bd, sem1, sem2):
    wid = _wid()
    nu_w = (nu + _NW - 1 - wid) // _NW

    def unit(g0, sup):
        pltpu.sync_copy(rowr_hbm.at[pl.ds(g0, sup)], irow.at[pl.ds(0, sup)])
        pltpu.sync_copy(colr_hbm.at[pl.ds(g0, sup)], icol.at[pl.ds(0, sup)])
        cps = []
        for j in range(sup):
            cps.append(pltpu.async_copy(
                x_hbm.at[irow.at[j]], bs.at[pl.ds(j * _GRP, _GRP)], sem1))
            cps.append(pltpu.async_copy(
                x_hbm.at[icol.at[j]], bd.at[pl.ds(j * _GRP, _GRP)], sem2))
        for cp in cps:
            cp.wait()
        e0 = g0 * _GRP
        pltpu.sync_copy(bs.at[pl.ds(0, sup * _GRP)],
                        xs_hbm.at[pl.ds(e0, sup * _GRP)])
        pltpu.sync_copy(bd.at[pl.ds(0, sup * _GRP)],
                        xd_hbm.at[pl.ds(e0, sup * _GRP)])

    def it_body(j, carry):
        unit((wid + _NW * j) * 8, 8)
        return carry

    lax.fori_loop(0, nu_w, it_body, 0)
    if tail:
        @pl.when(wid == 0)
        def _():
            unit(nu * 8, tail)


def _gather1_body(nu, tail, tab_hbm, rowr_hbm, out_hbm, irow, bs, sem1):
    wid = _wid()
    nu_w = (nu + _NW - 1 - wid) // _NW

    def unit(g0, sup):
        pltpu.sync_copy(rowr_hbm.at[pl.ds(g0, sup)], irow.at[pl.ds(0, sup)])
        cps = []
        for j in range(sup):
            cps.append(pltpu.async_copy(
                tab_hbm.at[irow.at[j]], bs.at[pl.ds(j * _GRP, _GRP)], sem1))
        for cp in cps:
            cp.wait()
        pltpu.sync_copy(bs.at[pl.ds(0, sup * _GRP)],
                        out_hbm.at[pl.ds(g0 * _GRP, sup * _GRP)])

    def it_body(j, carry):
        unit((wid + _NW * j) * 8, 8)
        return carry

    lax.fori_loop(0, nu_w, it_body, 0)
    if tail:
        @pl.when(wid == 0)
        def _():
            unit(nu * 8, tail)


def _tile_rows(n):
    big = -(-n // _NS)
    big = -(-big // 8) * 8
    last = n - big * (_NS - 1)
    assert last > 0
    return big, last


def _scatter_body(nu, tail, with_cnt, msg_hbm, colr_hbm, zz_hbm, *args):
    if with_cnt:
        (zc_hbm, ones_hbm, agg_hbm, cnt_hbm,
         icol, mbuf, ones_v, acc, accc, sem1, sem2) = args
    else:
        (agg_hbm, icol, mbuf, acc, sem1) = args
    c = lax.axis_index("c")
    s = lax.axis_index("s")
    wid = s * _NC + c
    n = acc.shape[0]
    big, last = _tile_rows(n)
    r0 = s * big

    def init(nrow):
        pltpu.sync_copy(zz_hbm.at[pl.ds(r0, nrow)], acc.at[pl.ds(r0, nrow)])
        if with_cnt:
            pltpu.sync_copy(zc_hbm.at[pl.ds(r0, nrow)],
                            accc.at[pl.ds(r0, nrow)])

    @pl.when(s < _NS - 1)
    def _():
        init(big)

    @pl.when(s == _NS - 1)
    def _():
        init(last)

    if with_cnt:
        pltpu.sync_copy(ones_hbm, ones_v)
    plsc.subcore_barrier()

    def unit(g0, sup):
        pltpu.sync_copy(colr_hbm.at[pl.ds(g0, sup)], icol.at[pl.ds(0, sup)])
        pltpu.sync_copy(msg_hbm.at[pl.ds(g0 * _GRP, sup * _GRP)],
                        mbuf.at[pl.ds(0, sup * _GRP)])
        cps = []
        for j in range(sup):
            cps.append(pltpu.async_copy(
                mbuf.at[pl.ds(j * _GRP, _GRP)], acc.at[icol.at[j]],
                sem1, add=True))
            if with_cnt:
                cps.append(pltpu.async_copy(
                    ones_v, accc.at[icol.at[j]], sem2, add=True))
        for cp in cps:
            cp.wait()

    def it_body(j, carry):
        unit((wid + _NW * j) * 8, 8)
        return carry

    nu_w = (nu + _NW - 1 - wid) // _NW
    lax.fori_loop(0, nu_w, it_body, 0)
    if tail:
        @pl.when(wid == 0)
        def _():
            unit(nu * 8, tail)

    plsc.subcore_barrier()

    def fini(nrow):
        pltpu.sync_copy(acc.at[pl.ds(r0, nrow)],
                        agg_hbm.at[c, pl.ds(r0, nrow)])
        if with_cnt:
            pltpu.sync_copy(accc.at[pl.ds(r0, nrow)],
                            cnt_hbm.at[c, pl.ds(r0, nrow)])

    @pl.when(s < _NS - 1)
    def _():
        fini(big)

    @pl.when(s == _NS - 1)
    def _():
        fini(last)


def _edge1_body(nk, xs_ref, xd_ref, m1_ref, w1_ref, mn1_ref, m2_ref, w2_ref,
                mn2_ref, g1k_ref, ek1_ref, msg_ref, gs2_ref):
    f32 = jnp.float32
    bf16 = jnp.bfloat16
    xs = xs_ref[...]
    ps = xs - xd_ref[...]
    sq = ps * ps
    e1 = (jnp.dot(ps, m1_ref[...], preferred_element_type=f32)
          - jnp.dot(sq, w1_ref[...], preferred_element_type=f32)
          - mn1_ref[...])
    g1 = jnp.exp(e1)
    e2 = (jnp.dot(ps, m2_ref[...], preferred_element_type=f32)
          - jnp.dot(sq, w2_ref[...], preferred_element_type=f32)
          - mn2_ref[...])
    gs2_ref[...] = jnp.exp(e2)
    xsb = xs.astype(bf16)
    wo = g1k_ref.shape[2]
    l = lax.broadcasted_iota(jnp.int32, (xs.shape[0], 128), 1)
    half = wo // 128
    for h in range(half):
        base = ((l >> 4) << 3) + 64 * h
        msg = jnp.zeros((xs.shape[0], 128), f32)
        for k in range(nk):
            bk = jnp.take_along_axis(g1, base + k, axis=1)
            msg = msg + bk * jnp.dot(
                xsb, g1k_ref[k, :, 128 * h:128 * (h + 1)],
                preferred_element_type=f32)
        msg_ref[:, 128 * h:128 * (h + 1)] = msg


def _edge2_body(nk, hs_ref, gs2_ref, g2k_ref, ek2_ref, msg2_ref):
    f32 = jnp.float32
    bf16 = jnp.bfloat16
    hs = hs_ref[...].astype(bf16)
    g2 = gs2_ref[...]
    l = lax.broadcasted_iota(jnp.int32, g2.shape, 1)
    base = (l >> 3) << 3
    msg = jnp.zeros((hs.shape[0], g2k_ref.shape[2]), f32)
    for k in range(nk):
        bk = jnp.take_along_axis(g2, base + k, axis=1)
        msg = msg + bk * jnp.dot(hs, g2k_ref[k], preferred_element_type=f32)
    msg2_ref[...] = msg


def _node1_body(agg_ref, cnt_ref, x_ref, r1t_ref, b1_ref, h_ref, invc_ref):
    f32 = jnp.float32
    agg = agg_ref[...]
    cnt = cnt_ref[...]
    asum = agg[0] + agg[1]
    csum = cnt[0] + cnt[1]
    ic = 1.0 / jnp.maximum(csum[:, 0:1], 1.0)
    a = (asum * ic + jnp.dot(x_ref[...], r1t_ref[...],
                             preferred_element_type=f32) + b1_ref[...])
    h_ref[...] = jnp.where(a > 0.0, a, jnp.exp(jnp.minimum(a, 0.0)) - 1.0)
    invc_ref[...] = ic


def _node2_body(agg2_ref, invc_ref, h_ref, r2t_ref, b2_ref, out_ref):
    f32 = jnp.float32
    a = agg2_ref[...]
    out_ref[...] = ((a[0] + a[1]) * invc_ref[...]
                    + jnp.dot(h_ref[...], r2t_ref[...],
                              preferred_element_type=f32)
                    + b2_ref[...])


def _blockdiag16(block):
    return jnp.kron(jnp.eye(16, dtype=block.dtype), block)


def kernel(x, edge_index, g1, mu1, sigma1, root1, b1, g2, mu2, sigma2, root2,
           b2):
    f32 = jnp.float32
    n, din = x.shape
    e = edge_index.shape[1]
    k, dim = mu1.shape
    hid = root1.shape[0]
    dout = root2.shape[0]
    ngrp = e // _GRP
    assert e % _GRP == 0

    row = edge_index[0]
    col = edge_index[1]
    rowr = row.reshape(ngrp, _GRP)
    colr = col.reshape(ngrp, _GRP)
    nu = ngrp // 8
    tail = ngrp % 8

    gathx = pl.kernel(
        functools.partial(_gather2_body, nu, tail),
        out_type=(jax.ShapeDtypeStruct((e, din), f32),
                  jax.ShapeDtypeStruct((e, din), f32)),
        mesh=_mesh(),
        compiler_params=_SC_PARAMS,
        scratch_types=(
            pltpu.VMEM((8, _GRP), jnp.int32),
            pltpu.VMEM((8, _GRP), jnp.int32),
            pltpu.VMEM((8 * _GRP, din), f32),
            pltpu.VMEM((8 * _GRP, din), f32),
            pltpu.SemaphoreType.DMA,
            pltpu.SemaphoreType.DMA,
        ),
    )
    xs, xd = gathx(x, rowr, colr)
    nrow16 = e // 16
    xs = xs.reshape(nrow16, 128)
    xd = xd.reshape(nrow16, 128)

    def gauss_mats(mu, sigma):
        w = 1.0 / (_EPS + sigma * sigma)
        mblk = jnp.zeros((8, 8), f32).at[:dim, :k].set((mu * w).T)
        wblk = jnp.zeros((8, 8), f32).at[:dim, :k].set((0.5 * w).T)
        mn = (0.5 * (mu * mu * w)).sum(axis=1)
        return (_blockdiag16(mblk), _blockdiag16(wblk),
                jnp.tile(mn, (16,))[None, :])

    m1m, w1m, mn1r = gauss_mats(mu1, sigma1)
    m2m, w2m, mn2r = gauss_mats(mu2, sigma2)
    bf16 = jnp.bfloat16
    g1r = g1.reshape(din, k, hid)
    g1k = jnp.stack([_blockdiag16(g1r[:, kk, :])
                     for kk in range(k)]).astype(bf16)
    g2r = g2.reshape(hid, k, dout)
    g2k = jnp.stack([_blockdiag16(g2r[:, kk, :])
                     for kk in range(k)]).astype(bf16)
    ek1 = jnp.stack([
        _blockdiag16(jnp.zeros((8, hid), f32).at[kk, :].set(1.0))
        for kk in range(k)]).astype(bf16)
    ek2 = jnp.stack([
        _blockdiag16(jnp.zeros((8, dout), f32).at[kk, :].set(1.0))
        for kk in range(k)]).astype(bf16)

    br = 2000
    full = lambda *s: pl.BlockSpec(s, lambda i: tuple(0 for _ in s))
    blk = lambda r, w: pl.BlockSpec((r, w), lambda i: (i, 0))
    msg1, gs2 = pl.pallas_call(
        functools.partial(_edge1_body, k),
        grid=(nrow16 // br,),
        in_specs=[blk(br, 128), blk(br, 128),
                  full(128, 128), full(128, 128), full(1, 128),
                  full(128, 128), full(128, 128), full(1, 128),
                  full(k, 128, 16 * hid), full(k, 128, 16 * hid)],
        out_specs=[blk(br, 16 * hid), blk(br, 128)],
        out_shape=[jax.ShapeDtypeStruct((nrow16, 16 * hid), f32),
                   jax.ShapeDtypeStruct((nrow16, 128), f32)],
    )(xs, xd, m1m, w1m, mn1r, m2m, w2m, mn2r, g1k, ek1)

    z16 = jnp.zeros((n, hid), f32)
    z8 = jnp.zeros((n, k), f32)
    ones8 = jnp.ones((_GRP, k), f32)
    scat1 = pl.kernel(
        functools.partial(_scatter_body, nu, tail, True),
        out_type=(jax.ShapeDtypeStruct((_NC, n, hid), f32),
                  jax.ShapeDtypeStruct((_NC, n, k), f32)),
        mesh=_mesh(),
        compiler_params=_SC_PARAMS,
        scratch_types=(
            pltpu.VMEM((8, _GRP), jnp.int32),
            pltpu.VMEM((8 * _GRP, hid), f32),
            pltpu.VMEM((_GRP, k), f32),
            pltpu.VMEM_SHARED((n, hid), f32),
            pltpu.VMEM_SHARED((n, k), f32),
            pltpu.SemaphoreType.DMA,
            pltpu.SemaphoreType.DMA,
        ),
    )
    agg1, cnt = scat1(msg1.reshape(e, hid), colr, z16, z8, ones8)

    bn = 5000
    h, invc = pl.pallas_call(
        _node1_body,
        grid=(n // bn,),
        in_specs=[
            pl.BlockSpec((_NC, bn, hid), lambda i: (0, i, 0)),
            pl.BlockSpec((_NC, bn, k), lambda i: (0, i, 0)),
            pl.BlockSpec((bn, din), lambda i: (i, 0)),
            full(din, hid), full(1, hid),
        ],
        out_specs=[pl.BlockSpec((bn, hid), lambda i: (i, 0)),
                   pl.BlockSpec((bn, 1), lambda i: (i, 0))],
        out_shape=[jax.ShapeDtypeStruct((n, hid), f32),
                   jax.ShapeDtypeStruct((n, 1), f32)],
    )(agg1, cnt, x, root1.T.astype(f32), b1[None, :].astype(f32))

    gathh = pl.kernel(
        functools.partial(_gather1_body, nu, tail),
        out_type=jax.ShapeDtypeStruct((e, hid), f32),
        mesh=_mesh(),
        compiler_params=_SC_PARAMS,
        scratch_types=(
            pltpu.VMEM((8, _GRP), jnp.int32),
            pltpu.VMEM((8 * _GRP, hid), f32),
            pltpu.SemaphoreType.DMA,
        ),
    )
    hs = gathh(h, rowr).reshape(nrow16, 16 * hid)

    msg2 = pl.pallas_call(
        functools.partial(_edge2_body, k),
        grid=(nrow16 // br,),
        in_specs=[blk(br, 16 * hid), blk(br, 128), full(k, 16 * hid, 128),
                  full(k, 128, 16 * dout)],
        out_specs=blk(br, 128),
        out_shape=jax.ShapeDtypeStruct((nrow16, 128), f32),
    )(hs, gs2, g2k, ek2)

    zo = jnp.zeros((n, dout), f32)
    scat2 = pl.kernel(
        functools.partial(_scatter_body, nu, tail, False),
        out_type=jax.ShapeDtypeStruct((_NC, n, dout), f32),
        mesh=_mesh(),
        compiler_params=_SC_PARAMS,
        scratch_types=(
            pltpu.VMEM((8, _GRP), jnp.int32),
            pltpu.VMEM((8 * _GRP, dout), f32),
            pltpu.VMEM_SHARED((n, dout), f32),
            pltpu.SemaphoreType.DMA,
        ),
    )
    agg2 = scat2(msg2.reshape(e, dout), colr, zo)

    out = pl.pallas_call(
        _node2_body,
        grid=(n // bn,),
        in_specs=[
            pl.BlockSpec((_NC, bn, dout), lambda i: (0, i, 0)),
            pl.BlockSpec((bn, 1), lambda i: (i, 0)),
            pl.BlockSpec((bn, hid), lambda i: (i, 0)),
            full(hid, dout), full(1, dout),
        ],
        out_specs=pl.BlockSpec((bn, dout), lambda i: (i, 0)),
        out_shape=jax.ShapeDtypeStruct((n, dout), f32),
    )(agg2, invc, h, root2.T.astype(f32), b2[None, :].astype(f32))
    return out


# speedup vs baseline: 13.2294x; 1.0223x over previous
import functools

import jax
import jax.numpy as jnp
from jax import lax
from jax.experimental import pallas as pl
from jax.experimental.pallas import tpu as pltpu
from jax.experimental.pallas import tpu_sc as plsc

_EPS = 1e-15
_NC = 2
_NS = 16
_NW = _NC * _NS
_GRP = 128


def _mesh():
    return plsc.VectorSubcoreMesh(core_axis_name="c", subcore_axis_name="s")


_SC_PARAMS = pltpu.CompilerParams(use_tc_tiling_on_sc=False)


def _wid():
    return lax.axis_index("s") * _NC + lax.axis_index("c")


def _gather2_body(nu, tail, x_hbm, rowr_hbm, colr_hbm, xs_hbm, xd_hbm,
                  irow, icol, bs, bd, sem1, sem2, semw):
    wid = _wid()
    nu_w = (nu + _NW - 1 - wid) // _NW
    hb = bs.shape[0] // 2

    def drain():
        pltpu.make_async_copy(
            xs_hbm.at[pl.ds(0, hb)], bs.at[pl.ds(0, hb)], semw).wait()
        pltpu.make_async_copy(
            xd_hbm.at[pl.ds(0, hb)], bd.at[pl.ds(0, hb)], semw).wait()

    def it_body(j, carry):
        g0 = (wid + _NW * j) * 8
        b0 = (j % 2) * hb
        pltpu.sync_copy(rowr_hbm.at[pl.ds(g0, 8)], irow)
        pltpu.sync_copy(colr_hbm.at[pl.ds(g0, 8)], icol)

        @pl.when(j >= 2)
        def _():
            drain()

        cps = []
        for t in range(8):
            cps.append(pltpu.async_copy(
                x_hbm.at[irow.at[t]], bs.at[pl.ds(b0 + t * _GRP, _GRP)],
                sem1))
            cps.append(pltpu.async_copy(
                x_hbm.at[icol.at[t]], bd.at[pl.ds(b0 + t * _GRP, _GRP)],
                sem2))
        for cp in cps:
            cp.wait()
        e0 = g0 * _GRP
        pltpu.async_copy(bs.at[pl.ds(b0, hb)],
                         xs_hbm.at[pl.ds(e0, hb)], semw)
        pltpu.async_copy(bd.at[pl.ds(b0, hb)],
                         xd_hbm.at[pl.ds(e0, hb)], semw)
        return carry

    lax.fori_loop(0, nu_w, it_body, 0)

    @pl.when(nu_w >= 2)
    def _():
        pltpu.make_async_copy(
            xs_hbm.at[pl.ds(0, hb)], bs.at[pl.ds(0, hb)], semw).wait()
        pltpu.make_async_copy(
            xd_hbm.at[pl.ds(0, hb)], bd.at[pl.ds(0, hb)], semw).wait()

    @pl.when(nu_w >= 1)
    def _():
        pltpu.make_async_copy(
            xs_hbm.at[pl.ds(0, hb)], bs.at[pl.ds(0, hb)], semw).wait()
        pltpu.make_async_copy(
            xd_hbm.at[pl.ds(0, hb)], bd.at[pl.ds(0, hb)], semw).wait()

    if tail:
        @pl.when(wid == 0)
        def _():
            g0 = nu * 8
            pltpu.sync_copy(rowr_hbm.at[pl.ds(g0, tail)],
                            irow.at[pl.ds(0, tail)])
            pltpu.sync_copy(colr_hbm.at[pl.ds(g0, tail)],
                            icol.at[pl.ds(0, tail)])
            cps = []
            for t in range(tail):
                cps.append(pltpu.async_copy(
                    x_hbm.at[irow.at[t]], bs.at[pl.ds(t * _GRP, _GRP)],
                    sem1))
                cps.append(pltpu.async_copy(
                    x_hbm.at[icol.at[t]], bd.at[pl.ds(t * _GRP, _GRP)],
                    sem2))
            for cp in cps:
                cp.wait()
            e0 = g0 * _GRP
            pltpu.sync_copy(bs.at[pl.ds(0, tail * _GRP)],
                            xs_hbm.at[pl.ds(e0, tail * _GRP)])
            pltpu.sync_copy(bd.at[pl.ds(0, tail * _GRP)],
                            xd_hbm.at[pl.ds(e0, tail * _GRP)])


def _gather1_body(nu, tail, tab_hbm, rowr_hbm, out_hbm, irow, bs, sem1,
                  semw):
    wid = _wid()
    nu_w = (nu + _NW - 1 - wid) // _NW
    hb = bs.shape[0] // 2

    def it_body(j, carry):
        g0 = (wid + _NW * j) * 8
        b0 = (j % 2) * hb
        pltpu.sync_copy(rowr_hbm.at[pl.ds(g0, 8)], irow)

        @pl.when(j >= 2)
        def _():
            pltpu.make_async_copy(
                out_hbm.at[pl.ds(0, hb)], bs.at[pl.ds(0, hb)], semw).wait()

        cps = []
        for t in range(8):
            cps.append(pltpu.async_copy(
                tab_hbm.at[irow.at[t]], bs.at[pl.ds(b0 + t * _GRP, _GRP)],
                sem1))
        for cp in cps:
            cp.wait()
        pltpu.async_copy(bs.at[pl.ds(b0, hb)],
                         out_hbm.at[pl.ds(g0 * _GRP, hb)], semw)
        return carry

    lax.fori_loop(0, nu_w, it_body, 0)

    @pl.when(nu_w >= 2)
    def _():
        pltpu.make_async_copy(
            out_hbm.at[pl.ds(0, hb)], bs.at[pl.ds(0, hb)], semw).wait()

    @pl.when(nu_w >= 1)
    def _():
        pltpu.make_async_copy(
            out_hbm.at[pl.ds(0, hb)], bs.at[pl.ds(0, hb)], semw).wait()

    if tail:
        @pl.when(wid == 0)
        def _():
            g0 = nu * 8
            pltpu.sync_copy(rowr_hbm.at[pl.ds(g0, tail)],
                            irow.at[pl.ds(0, tail)])
            cps = []
            for t in range(tail):
                cps.append(pltpu.async_copy(
                    tab_hbm.at[irow.at[t]], bs.at[pl.ds(t * _GRP, _GRP)],
                    sem1))
            for cp in cps:
                cp.wait()
            pltpu.sync_copy(bs.at[pl.ds(0, tail * _GRP)],
                            out_hbm.at[pl.ds(g0 * _GRP, tail * _GRP)])


def _tile_rows(n):
    big = -(-n // _NS)
    big = -(-big // 8) * 8
    last = n - big * (_NS - 1)
    assert last > 0
    return big, last


def _scatter_body(nu, tail, with_cnt, msg_hbm, colr_hbm, zz_hbm, *args):
    if with_cnt:
        (zc_hbm, ones_hbm, agg_hbm, cnt_hbm,
         icol, mbuf, ones_v, acc, accc, sem1, sem2) = args
    else:
        (agg_hbm, icol, mbuf, acc, sem1) = args
    c = lax.axis_index("c")
    s = lax.axis_index("s")
    wid = s * _NC + c
    n = acc.shape[0]
    big, last = _tile_rows(n)
    r0 = s * big

    def init(nrow):
        pltpu.sync_copy(zz_hbm.at[pl.ds(r0, nrow)], acc.at[pl.ds(r0, nrow)])
        if with_cnt:
            pltpu.sync_copy(zc_hbm.at[pl.ds(r0, nrow)],
                            accc.at[pl.ds(r0, nrow)])

    @pl.when(s < _NS - 1)
    def _():
        init(big)

    @pl.when(s == _NS - 1)
    def _():
        init(last)

    if with_cnt:
        pltpu.sync_copy(ones_hbm, ones_v)
    plsc.subcore_barrier()

    def unit(g0, sup):
        pltpu.sync_copy(colr_hbm.at[pl.ds(g0, sup)], icol.at[pl.ds(0, sup)])
        pltpu.sync_copy(msg_hbm.at[pl.ds(g0 * _GRP, sup * _GRP)],
                        mbuf.at[pl.ds(0, sup * _GRP)])
        cps = []
        for j in range(sup):
            cps.append(pltpu.async_copy(
                mbuf.at[pl.ds(j * _GRP, _GRP)], acc.at[icol.at[j]],
                sem1, add=True))
            if with_cnt:
                cps.append(pltpu.async_copy(
                    ones_v, accc.at[icol.at[j]], sem2, add=True))
        for cp in cps:
            cp.wait()

    def it_body(j, carry):
        unit((wid + _NW * j) * 8, 8)
        return carry

    nu_w = (nu + _NW - 1 - wid) // _NW
    lax.fori_loop(0, nu_w, it_body, 0)
    if tail:
        @pl.when(wid == 0)
        def _():
            unit(nu * 8, tail)

    plsc.subcore_barrier()

    def fini(nrow):
        pltpu.sync_copy(acc.at[pl.ds(r0, nrow)],
                        agg_hbm.at[c, pl.ds(r0, nrow)])
        if with_cnt:
            pltpu.sync_copy(accc.at[pl.ds(r0, nrow)],
                            cnt_hbm.at[c, pl.ds(r0, nrow)])

    @pl.when(s < _NS - 1)
    def _():
        fini(big)

    @pl.when(s == _NS - 1)
    def _():
        fini(last)


def _edge1_body(nk, xs_ref, xd_ref, m1_ref, w1_ref, mn1_ref, m2_ref, w2_ref,
                mn2_ref, g1k_ref, ek1_ref, msg_ref, gs2_ref):
    f32 = jnp.float32
    bf16 = jnp.bfloat16
    xs = xs_ref[...]
    ps = xs - xd_ref[...]
    sq = ps * ps
    e1 = (jnp.dot(ps, m1_ref[...], preferred_element_type=f32)
          - jnp.dot(sq, w1_ref[...], preferred_element_type=f32)
          - mn1_ref[...])
    g1 = jnp.exp(e1)
    e2 = (jnp.dot(ps, m2_ref[...], preferred_element_type=f32)
          - jnp.dot(sq, w2_ref[...], preferred_element_type=f32)
          - mn2_ref[...])
    gs2_ref[...] = jnp.exp(e2)
    xsb = xs.astype(bf16)
    wo = g1k_ref.shape[2]
    l = lax.broadcasted_iota(jnp.int32, (xs.shape[0], 128), 1)
    half = wo // 128
    for h in range(half):
        base = ((l >> 4) << 3) + 64 * h
        msg = jnp.zeros((xs.shape[0], 128), f32)
        for k in range(nk):
            bk = jnp.take_along_axis(g1, base + k, axis=1)
            msg = msg + bk * jnp.dot(
                xsb, g1k_ref[k, :, 128 * h:128 * (h + 1)],
                preferred_element_type=f32)
        msg_ref[:, 128 * h:128 * (h + 1)] = msg


def _edge2_body(nk, hs_ref, gs2_ref, g2k_ref, ek2_ref, msg2_ref):
    f32 = jnp.float32
    bf16 = jnp.bfloat16
    hs = hs_ref[...].astype(bf16)
    g2 = gs2_ref[...]
    l = lax.broadcasted_iota(jnp.int32, g2.shape, 1)
    base = (l >> 3) << 3
    msg = jnp.zeros((hs.shape[0], g2k_ref.shape[2]), f32)
    for k in range(nk):
        bk = jnp.take_along_axis(g2, base + k, axis=1)
        msg = msg + bk * jnp.dot(hs, g2k_ref[k], preferred_element_type=f32)
    msg2_ref[...] = msg


def _node1_body(agg_ref, cnt_ref, x_ref, r1t_ref, b1_ref, h_ref, invc_ref):
    f32 = jnp.float32
    agg = agg_ref[...]
    cnt = cnt_ref[...]
    asum = agg[0] + agg[1]
    csum = cnt[0] + cnt[1]
    ic = 1.0 / jnp.maximum(csum[:, 0:1], 1.0)
    a = (asum * ic + jnp.dot(x_ref[...], r1t_ref[...],
                             preferred_element_type=f32) + b1_ref[...])
    h_ref[...] = jnp.where(a > 0.0, a, jnp.exp(jnp.minimum(a, 0.0)) - 1.0)
    invc_ref[...] = ic


def _node2_body(agg2_ref, invc_ref, h_ref, r2t_ref, b2_ref, out_ref):
    f32 = jnp.float32
    a = agg2_ref[...]
    out_ref[...] = ((a[0] + a[1]) * invc_ref[...]
                    + jnp.dot(h_ref[...], r2t_ref[...],
                              preferred_element_type=f32)
                    + b2_ref[...])


def _blockdiag16(block):
    return jnp.kron(jnp.eye(16, dtype=block.dtype), block)


def kernel(x, edge_index, g1, mu1, sigma1, root1, b1, g2, mu2, sigma2, root2,
           b2):
    f32 = jnp.float32
    n, din = x.shape
    e = edge_index.shape[1]
    k, dim = mu1.shape
    hid = root1.shape[0]
    dout = root2.shape[0]
    ngrp = e // _GRP
    assert e % _GRP == 0

    row = edge_index[0]
    col = edge_index[1]
    rowr = row.reshape(ngrp, _GRP)
    colr = col.reshape(ngrp, _GRP)
    nu = ngrp // 8
    tail = ngrp % 8

    gathx = pl.kernel(
        functools.partial(_gather2_body, nu, tail),
        out_type=(jax.ShapeDtypeStruct((e, din), f32),
                  jax.ShapeDtypeStruct((e, din), f32)),
        mesh=_mesh(),
        compiler_params=_SC_PARAMS,
        scratch_types=(
            pltpu.VMEM((8, _GRP), jnp.int32),
            pltpu.VMEM((8, _GRP), jnp.int32),
            pltpu.VMEM((16 * _GRP, din), f32),
            pltpu.VMEM((16 * _GRP, din), f32),
            pltpu.SemaphoreType.DMA,
            pltpu.SemaphoreType.DMA,
            pltpu.SemaphoreType.DMA,
        ),
    )
    xs, xd = gathx(x, rowr, colr)
    nrow16 = e // 16
    xs = xs.reshape(nrow16, 128)
    xd = xd.reshape(nrow16, 128)

    def gauss_mats(mu, sigma):
        w = 1.0 / (_EPS + sigma * sigma)
        mblk = jnp.zeros((8, 8), f32).at[:dim, :k].set((mu * w).T)
        wblk = jnp.zeros((8, 8), f32).at[:dim, :k].set((0.5 * w).T)
        mn = (0.5 * (mu * mu * w)).sum(axis=1)
        return (_blockdiag16(mblk), _blockdiag16(wblk),
                jnp.tile(mn, (16,))[None, :])

    m1m, w1m, mn1r = gauss_mats(mu1, sigma1)
    m2m, w2m, mn2r = gauss_mats(mu2, sigma2)
    bf16 = jnp.bfloat16
    g1r = g1.reshape(din, k, hid)
    g1k = jnp.stack([_blockdiag16(g1r[:, kk, :])
                     for kk in range(k)]).astype(bf16)
    g2r = g2.reshape(hid, k, dout)
    g2k = jnp.stack([_blockdiag16(g2r[:, kk, :])
                     for kk in range(k)]).astype(bf16)
    ek1 = jnp.stack([
        _blockdiag16(jnp.zeros((8, hid), f32).at[kk, :].set(1.0))
        for kk in range(k)]).astype(bf16)
    ek2 = jnp.stack([
        _blockdiag16(jnp.zeros((8, dout), f32).at[kk, :].set(1.0))
        for kk in range(k)]).astype(bf16)

    br = 2000
    full = lambda *s: pl.BlockSpec(s, lambda i: tuple(0 for _ in s))
    blk = lambda r, w: pl.BlockSpec((r, w), lambda i: (i, 0))
    msg1, gs2 = pl.pallas_call(
        functools.partial(_edge1_body, k),
        grid=(nrow16 // br,),
        in_specs=[blk(br, 128), blk(br, 128),
                  full(128, 128), full(128, 128), full(1, 128),
                  full(128, 128), full(128, 128), full(1, 128),
                  full(k, 128, 16 * hid), full(k, 128, 16 * hid)],
        out_specs=[blk(br, 16 * hid), blk(br, 128)],
        out_shape=[jax.ShapeDtypeStruct((nrow16, 16 * hid), f32),
                   jax.ShapeDtypeStruct((nrow16, 128), f32)],
    )(xs, xd, m1m, w1m, mn1r, m2m, w2m, mn2r, g1k, ek1)

    z16 = jnp.zeros((n, hid), f32)
    z8 = jnp.zeros((n, k), f32)
    ones8 = jnp.ones((_GRP, k), f32)
    scat1 = pl.kernel(
        functools.partial(_scatter_body, nu, tail, True),
        out_type=(jax.ShapeDtypeStruct((_NC, n, hid), f32),
                  jax.ShapeDtypeStruct((_NC, n, k), f32)),
        mesh=_mesh(),
        compiler_params=_SC_PARAMS,
        scratch_types=(
            pltpu.VMEM((8, _GRP), jnp.int32),
            pltpu.VMEM((8 * _GRP, hid), f32),
            pltpu.VMEM((_GRP, k), f32),
            pltpu.VMEM_SHARED((n, hid), f32),
            pltpu.VMEM_SHARED((n, k), f32),
            pltpu.SemaphoreType.DMA,
            pltpu.SemaphoreType.DMA,
        ),
    )
    agg1, cnt = scat1(msg1.reshape(e, hid), colr, z16, z8, ones8)

    bn = 5000
    h, invc = pl.pallas_call(
        _node1_body,
        grid=(n // bn,),
        in_specs=[
            pl.BlockSpec((_NC, bn, hid), lambda i: (0, i, 0)),
            pl.BlockSpec((_NC, bn, k), lambda i: (0, i, 0)),
            pl.BlockSpec((bn, din), lambda i: (i, 0)),
            full(din, hid), full(1, hid),
        ],
        out_specs=[pl.BlockSpec((bn, hid), lambda i: (i, 0)),
                   pl.BlockSpec((bn, 1), lambda i: (i, 0))],
        out_shape=[jax.ShapeDtypeStruct((n, hid), f32),
                   jax.ShapeDtypeStruct((n, 1), f32)],
    )(agg1, cnt, x, root1.T.astype(f32), b1[None, :].astype(f32))

    gathh = pl.kernel(
        functools.partial(_gather1_body, nu, tail),
        out_type=jax.ShapeDtypeStruct((e, hid), f32),
        mesh=_mesh(),
        compiler_params=_SC_PARAMS,
        scratch_types=(
            pltpu.VMEM((8, _GRP), jnp.int32),
            pltpu.VMEM((16 * _GRP, hid), f32),
            pltpu.SemaphoreType.DMA,
            pltpu.SemaphoreType.DMA,
        ),
    )
    hs = gathh(h, rowr).reshape(nrow16, 16 * hid)

    msg2 = pl.pallas_call(
        functools.partial(_edge2_body, k),
        grid=(nrow16 // br,),
        in_specs=[blk(br, 16 * hid), blk(br, 128), full(k, 16 * hid, 128),
                  full(k, 128, 16 * dout)],
        out_specs=blk(br, 128),
        out_shape=jax.ShapeDtypeStruct((nrow16, 128), f32),
    )(hs, gs2, g2k, ek2)

    zo = jnp.zeros((n, dout), f32)
    scat2 = pl.kernel(
        functools.partial(_scatter_body, nu, tail, False),
        out_type=jax.ShapeDtypeStruct((_NC, n, dout), f32),
        mesh=_mesh(),
        compiler_params=_SC_PARAMS,
        scratch_types=(
            pltpu.VMEM((8, _GRP), jnp.int32),
            pltpu.VMEM((8 * _GRP, dout), f32),
            pltpu.VMEM_SHARED((n, dout), f32),
            pltpu.SemaphoreType.DMA,
        ),
    )
    agg2 = scat2(msg2.reshape(e, dout), colr, zo)

    out = pl.pallas_call(
        _node2_body,
        grid=(n // bn,),
        in_specs=[
            pl.BlockSpec((_NC, bn, dout), lambda i: (0, i, 0)),
            pl.BlockSpec((bn, 1), lambda i: (i, 0)),
            pl.BlockSpec((bn, hid), lambda i: (i, 0)),
            full(hid, dout), full(1, dout),
        ],
        out_specs=pl.BlockSpec((bn, dout), lambda i: (i, 0)),
        out_shape=jax.ShapeDtypeStruct((n, dout), f32),
    )(agg2, invc, h, root2.T.astype(f32), b2[None, :].astype(f32))
    return out


# speedup vs baseline: 14.0243x; 1.0601x over previous
import functools

import jax
import jax.numpy as jnp
from jax import lax
from jax.experimental import pallas as pl
from jax.experimental.pallas import tpu as pltpu
from jax.experimental.pallas import tpu_sc as plsc

_EPS = 1e-15
_NC = 2
_NS = 16
_NW = _NC * _NS
_GRP = 128


def _mesh():
    return plsc.VectorSubcoreMesh(core_axis_name="c", subcore_axis_name="s")


_SC_PARAMS = pltpu.CompilerParams(use_tc_tiling_on_sc=False)


def _wid():
    return lax.axis_index("s") * _NC + lax.axis_index("c")


def _gather2_body(nu, tail, x_hbm, rowr_hbm, colr_hbm, xs_hbm, xd_hbm,
                  irow, icol, bs, bd, sem1, sem2, semw):
    wid = _wid()
    nu_w = (nu + _NW - 1 - wid) // _NW
    hb = bs.shape[0] // 2

    def drain():
        pltpu.make_async_copy(
            xs_hbm.at[pl.ds(0, hb)], bs.at[pl.ds(0, hb)], semw).wait()
        pltpu.make_async_copy(
            xd_hbm.at[pl.ds(0, hb)], bd.at[pl.ds(0, hb)], semw).wait()

    def it_body(j, carry):
        g0 = (wid + _NW * j) * 8
        b0 = (j % 2) * hb
        pltpu.sync_copy(rowr_hbm.at[pl.ds(g0, 8)], irow)
        pltpu.sync_copy(colr_hbm.at[pl.ds(g0, 8)], icol)

        @pl.when(j >= 2)
        def _():
            drain()

        cps = []
        for t in range(8):
            cps.append(pltpu.async_copy(
                x_hbm.at[irow.at[t]], bs.at[pl.ds(b0 + t * _GRP, _GRP)],
                sem1))
            cps.append(pltpu.async_copy(
                x_hbm.at[icol.at[t]], bd.at[pl.ds(b0 + t * _GRP, _GRP)],
                sem2))
        for cp in cps:
            cp.wait()
        e0 = g0 * _GRP
        pltpu.async_copy(bs.at[pl.ds(b0, hb)],
                         xs_hbm.at[pl.ds(e0, hb)], semw)
        pltpu.async_copy(bd.at[pl.ds(b0, hb)],
                         xd_hbm.at[pl.ds(e0, hb)], semw)
        return carry

    lax.fori_loop(0, nu_w, it_body, 0)

    @pl.when(nu_w >= 2)
    def _():
        pltpu.make_async_copy(
            xs_hbm.at[pl.ds(0, hb)], bs.at[pl.ds(0, hb)], semw).wait()
        pltpu.make_async_copy(
            xd_hbm.at[pl.ds(0, hb)], bd.at[pl.ds(0, hb)], semw).wait()

    @pl.when(nu_w >= 1)
    def _():
        pltpu.make_async_copy(
            xs_hbm.at[pl.ds(0, hb)], bs.at[pl.ds(0, hb)], semw).wait()
        pltpu.make_async_copy(
            xd_hbm.at[pl.ds(0, hb)], bd.at[pl.ds(0, hb)], semw).wait()

    if tail:
        @pl.when(wid == 0)
        def _():
            g0 = nu * 8
            pltpu.sync_copy(rowr_hbm.at[pl.ds(g0, tail)],
                            irow.at[pl.ds(0, tail)])
            pltpu.sync_copy(colr_hbm.at[pl.ds(g0, tail)],
                            icol.at[pl.ds(0, tail)])
            cps = []
            for t in range(tail):
                cps.append(pltpu.async_copy(
                    x_hbm.at[irow.at[t]], bs.at[pl.ds(t * _GRP, _GRP)],
                    sem1))
                cps.append(pltpu.async_copy(
                    x_hbm.at[icol.at[t]], bd.at[pl.ds(t * _GRP, _GRP)],
                    sem2))
            for cp in cps:
                cp.wait()
            e0 = g0 * _GRP
            pltpu.sync_copy(bs.at[pl.ds(0, tail * _GRP)],
                            xs_hbm.at[pl.ds(e0, tail * _GRP)])
            pltpu.sync_copy(bd.at[pl.ds(0, tail * _GRP)],
                            xd_hbm.at[pl.ds(e0, tail * _GRP)])


def _gather1_body(nu, tail, tab_hbm, rowr_hbm, out_hbm, irow, bs, sem1,
                  semw):
    wid = _wid()
    nu_w = (nu + _NW - 1 - wid) // _NW
    hb = bs.shape[0] // 2

    def it_body(j, carry):
        g0 = (wid + _NW * j) * 8
        b0 = (j % 2) * hb
        pltpu.sync_copy(rowr_hbm.at[pl.ds(g0, 8)], irow)

        @pl.when(j >= 2)
        def _():
            pltpu.make_async_copy(
                out_hbm.at[pl.ds(0, hb)], bs.at[pl.ds(0, hb)], semw).wait()

        cps = []
        for t in range(8):
            cps.append(pltpu.async_copy(
                tab_hbm.at[irow.at[t]], bs.at[pl.ds(b0 + t * _GRP, _GRP)],
                sem1))
        for cp in cps:
            cp.wait()
        pltpu.async_copy(bs.at[pl.ds(b0, hb)],
                         out_hbm.at[pl.ds(g0 * _GRP, hb)], semw)
        return carry

    lax.fori_loop(0, nu_w, it_body, 0)

    @pl.when(nu_w >= 2)
    def _():
        pltpu.make_async_copy(
            out_hbm.at[pl.ds(0, hb)], bs.at[pl.ds(0, hb)], semw).wait()

    @pl.when(nu_w >= 1)
    def _():
        pltpu.make_async_copy(
            out_hbm.at[pl.ds(0, hb)], bs.at[pl.ds(0, hb)], semw).wait()

    if tail:
        @pl.when(wid == 0)
        def _():
            g0 = nu * 8
            pltpu.sync_copy(rowr_hbm.at[pl.ds(g0, tail)],
                            irow.at[pl.ds(0, tail)])
            cps = []
            for t in range(tail):
                cps.append(pltpu.async_copy(
                    tab_hbm.at[irow.at[t]], bs.at[pl.ds(t * _GRP, _GRP)],
                    sem1))
            for cp in cps:
                cp.wait()
            pltpu.sync_copy(bs.at[pl.ds(0, tail * _GRP)],
                            out_hbm.at[pl.ds(g0 * _GRP, tail * _GRP)])


def _tile_rows(n):
    big = -(-n // _NS)
    big = -(-big // 8) * 8
    last = n - big * (_NS - 1)
    assert last > 0
    return big, last


def _scatter_body(nu, tail, with_cnt, msg_hbm, colr_hbm, zz_hbm, *args):
    if with_cnt:
        (zc_hbm, ones_hbm, agg_hbm, cnt_hbm,
         icol, mbuf, ones_v, acc, accc, sem1, sem2) = args
    else:
        (agg_hbm, icol, mbuf, acc, sem1, sem2) = args
    c = lax.axis_index("c")
    s = lax.axis_index("s")
    wid = s * _NC + c
    n = acc.shape[0]
    big, last = _tile_rows(n)
    r0 = s * big

    def init(nrow):
        pltpu.sync_copy(zz_hbm.at[pl.ds(r0, nrow)], acc.at[pl.ds(r0, nrow)])
        if with_cnt:
            pltpu.sync_copy(zc_hbm.at[pl.ds(r0, nrow)],
                            accc.at[pl.ds(r0, nrow)])

    @pl.when(s < _NS - 1)
    def _():
        init(big)

    @pl.when(s == _NS - 1)
    def _():
        init(last)

    if with_cnt:
        pltpu.sync_copy(ones_hbm, ones_v)
    plsc.subcore_barrier()
    nu_w = (nu + _NW - 1 - wid) // _NW

    def load(j, b):
        g0 = (wid + _NW * j) * 8
        cl1 = pltpu.async_copy(colr_hbm.at[pl.ds(g0, 8)],
                               icol.at[pl.ds(b * 8, 8)], sem2)
        cl2 = pltpu.async_copy(msg_hbm.at[pl.ds(g0 * _GRP, 8 * _GRP)],
                               mbuf.at[pl.ds(b * 8 * _GRP, 8 * _GRP)], sem2)
        return cl1, cl2

    @pl.when(nu_w >= 1)
    def _():
        for cp in load(0, 0):
            cp.wait()

    def it_body(j, carry):
        b = j % 2
        boff = b * 8 * _GRP
        cps = []
        for t in range(8):
            cps.append(pltpu.async_copy(
                mbuf.at[pl.ds(boff + t * _GRP, _GRP)],
                acc.at[icol.at[b * 8 + t]], sem1, add=True))
            if with_cnt:
                cps.append(pltpu.async_copy(
                    ones_v, accc.at[icol.at[b * 8 + t]], sem1, add=True))

        @pl.when(j + 1 < nu_w)
        def _():
            for cp in load(j + 1, 1 - b):
                cp.wait()

        for cp in cps:
            cp.wait()
        return carry

    lax.fori_loop(0, nu_w, it_body, 0)
    if tail:
        @pl.when(wid == 0)
        def _():
            g0 = nu * 8
            pltpu.sync_copy(colr_hbm.at[pl.ds(g0, tail)],
                            icol.at[pl.ds(0, tail)])
            pltpu.sync_copy(msg_hbm.at[pl.ds(g0 * _GRP, tail * _GRP)],
                            mbuf.at[pl.ds(0, tail * _GRP)])
            cps = []
            for t in range(tail):
                cps.append(pltpu.async_copy(
                    mbuf.at[pl.ds(t * _GRP, _GRP)], acc.at[icol.at[t]],
                    sem1, add=True))
                if with_cnt:
                    cps.append(pltpu.async_copy(
                        ones_v, accc.at[icol.at[t]], sem1, add=True))
            for cp in cps:
                cp.wait()

    plsc.subcore_barrier()

    def fini(nrow):
        pltpu.sync_copy(acc.at[pl.ds(r0, nrow)],
                        agg_hbm.at[c, pl.ds(r0, nrow)])
        if with_cnt:
            pltpu.sync_copy(accc.at[pl.ds(r0, nrow)],
                            cnt_hbm.at[c, pl.ds(r0, nrow)])

    @pl.when(s < _NS - 1)
    def _():
        fini(big)

    @pl.when(s == _NS - 1)
    def _():
        fini(last)


def _edge1_body(nk, xs_ref, xd_ref, m1_ref, w1_ref, mn1_ref, m2_ref, w2_ref,
                mn2_ref, g1k_ref, ek1_ref, msg_ref, gs2_ref):
    f32 = jnp.float32
    bf16 = jnp.bfloat16
    xs = xs_ref[...]
    ps = xs - xd_ref[...]
    sq = ps * ps
    e1 = (jnp.dot(ps, m1_ref[...], preferred_element_type=f32)
          - jnp.dot(sq, w1_ref[...], preferred_element_type=f32)
          - mn1_ref[...])
    g1 = jnp.exp(e1)
    e2 = (jnp.dot(ps, m2_ref[...], preferred_element_type=f32)
          - jnp.dot(sq, w2_ref[...], preferred_element_type=f32)
          - mn2_ref[...])
    gs2_ref[...] = jnp.exp(e2)
    xsb = xs.astype(bf16)
    wo = g1k_ref.shape[2]
    l = lax.broadcasted_iota(jnp.int32, (xs.shape[0], 128), 1)
    half = wo // 128
    for h in range(half):
        base = ((l >> 4) << 3) + 64 * h
        msg = jnp.zeros((xs.shape[0], 128), f32)
        for k in range(nk):
            bk = jnp.take_along_axis(g1, base + k, axis=1)
            msg = msg + bk * jnp.dot(
                xsb, g1k_ref[k, :, 128 * h:128 * (h + 1)],
                preferred_element_type=f32)
        msg_ref[:, 128 * h:128 * (h + 1)] = msg


def _edge2_body(nk, hs_ref, gs2_ref, g2k_ref, ek2_ref, msg2_ref):
    f32 = jnp.float32
    bf16 = jnp.bfloat16
    hs = hs_ref[...].astype(bf16)
    g2 = gs2_ref[...]
    l = lax.broadcasted_iota(jnp.int32, g2.shape, 1)
    base = (l >> 3) << 3
    msg = jnp.zeros((hs.shape[0], g2k_ref.shape[2]), f32)
    for k in range(nk):
        bk = jnp.take_along_axis(g2, base + k, axis=1)
        msg = msg + bk * jnp.dot(hs, g2k_ref[k], preferred_element_type=f32)
    msg2_ref[...] = msg


def _node1_body(agg_ref, cnt_ref, x_ref, r1t_ref, b1_ref, h_ref, invc_ref):
    f32 = jnp.float32
    agg = agg_ref[...]
    cnt = cnt_ref[...]
    asum = agg[0] + agg[1]
    csum = cnt[0] + cnt[1]
    ic = 1.0 / jnp.maximum(csum[:, 0:1], 1.0)
    a = (asum * ic + jnp.dot(x_ref[...], r1t_ref[...],
                             preferred_element_type=f32) + b1_ref[...])
    h_ref[...] = jnp.where(a > 0.0, a, jnp.exp(jnp.minimum(a, 0.0)) - 1.0)
    invc_ref[...] = ic


def _node2_body(agg2_ref, invc_ref, h_ref, r2t_ref, b2_ref, out_ref):
    f32 = jnp.float32
    a = agg2_ref[...]
    out_ref[...] = ((a[0] + a[1]) * invc_ref[...]
                    + jnp.dot(h_ref[...], r2t_ref[...],
                              preferred_element_type=f32)
                    + b2_ref[...])


def _blockdiag16(block):
    return jnp.kron(jnp.eye(16, dtype=block.dtype), block)


def kernel(x, edge_index, g1, mu1, sigma1, root1, b1, g2, mu2, sigma2, root2,
           b2):
    f32 = jnp.float32
    n, din = x.shape
    e = edge_index.shape[1]
    k, dim = mu1.shape
    hid = root1.shape[0]
    dout = root2.shape[0]
    ngrp = e // _GRP
    assert e % _GRP == 0

    row = edge_index[0]
    col = edge_index[1]
    rowr = row.reshape(ngrp, _GRP)
    colr = col.reshape(ngrp, _GRP)
    nu = ngrp // 8
    tail = ngrp % 8

    gathx = pl.kernel(
        functools.partial(_gather2_body, nu, tail),
        out_type=(jax.ShapeDtypeStruct((e, din), f32),
                  jax.ShapeDtypeStruct((e, din), f32)),
        mesh=_mesh(),
        compiler_params=_SC_PARAMS,
        scratch_types=(
            pltpu.VMEM((8, _GRP), jnp.int32),
            pltpu.VMEM((8, _GRP), jnp.int32),
            pltpu.VMEM((16 * _GRP, din), f32),
            pltpu.VMEM((16 * _GRP, din), f32),
            pltpu.SemaphoreType.DMA,
            pltpu.SemaphoreType.DMA,
            pltpu.SemaphoreType.DMA,
        ),
    )
    xs, xd = gathx(x, rowr, colr)
    nrow16 = e // 16
    xs = xs.reshape(nrow16, 128)
    xd = xd.reshape(nrow16, 128)

    def gauss_mats(mu, sigma):
        w = 1.0 / (_EPS + sigma * sigma)
        mblk = jnp.zeros((8, 8), f32).at[:dim, :k].set((mu * w).T)
        wblk = jnp.zeros((8, 8), f32).at[:dim, :k].set((0.5 * w).T)
        mn = (0.5 * (mu * mu * w)).sum(axis=1)
        return (_blockdiag16(mblk), _blockdiag16(wblk),
                jnp.tile(mn, (16,))[None, :])

    m1m, w1m, mn1r = gauss_mats(mu1, sigma1)
    m2m, w2m, mn2r = gauss_mats(mu2, sigma2)
    bf16 = jnp.bfloat16
    g1r = g1.reshape(din, k, hid)
    g1k = jnp.stack([_blockdiag16(g1r[:, kk, :])
                     for kk in range(k)]).astype(bf16)
    g2r = g2.reshape(hid, k, dout)
    g2k = jnp.stack([_blockdiag16(g2r[:, kk, :])
                     for kk in range(k)]).astype(bf16)
    ek1 = jnp.stack([
        _blockdiag16(jnp.zeros((8, hid), f32).at[kk, :].set(1.0))
        for kk in range(k)]).astype(bf16)
    ek2 = jnp.stack([
        _blockdiag16(jnp.zeros((8, dout), f32).at[kk, :].set(1.0))
        for kk in range(k)]).astype(bf16)

    br = 2000
    full = lambda *s: pl.BlockSpec(s, lambda i: tuple(0 for _ in s))
    blk = lambda r, w: pl.BlockSpec((r, w), lambda i: (i, 0))
    msg1, gs2 = pl.pallas_call(
        functools.partial(_edge1_body, k),
        grid=(nrow16 // br,),
        in_specs=[blk(br, 128), blk(br, 128),
                  full(128, 128), full(128, 128), full(1, 128),
                  full(128, 128), full(128, 128), full(1, 128),
                  full(k, 128, 16 * hid), full(k, 128, 16 * hid)],
        out_specs=[blk(br, 16 * hid), blk(br, 128)],
        out_shape=[jax.ShapeDtypeStruct((nrow16, 16 * hid), f32),
                   jax.ShapeDtypeStruct((nrow16, 128), f32)],
    )(xs, xd, m1m, w1m, mn1r, m2m, w2m, mn2r, g1k, ek1)

    z16 = jnp.zeros((n, hid), f32)
    z8 = jnp.zeros((n, k), f32)
    ones8 = jnp.ones((_GRP, k), f32)
    scat1 = pl.kernel(
        functools.partial(_scatter_body, nu, tail, True),
        out_type=(jax.ShapeDtypeStruct((_NC, n, hid), f32),
                  jax.ShapeDtypeStruct((_NC, n, k), f32)),
        mesh=_mesh(),
        compiler_params=_SC_PARAMS,
        scratch_types=(
            pltpu.VMEM((16, _GRP), jnp.int32),
            pltpu.VMEM((16 * _GRP, hid), f32),
            pltpu.VMEM((_GRP, k), f32),
            pltpu.VMEM_SHARED((n, hid), f32),
            pltpu.VMEM_SHARED((n, k), f32),
            pltpu.SemaphoreType.DMA,
            pltpu.SemaphoreType.DMA,
        ),
    )
    agg1, cnt = scat1(msg1.reshape(e, hid), colr, z16, z8, ones8)

    bn = 5000
    h, invc = pl.pallas_call(
        _node1_body,
        grid=(n // bn,),
        in_specs=[
            pl.BlockSpec((_NC, bn, hid), lambda i: (0, i, 0)),
            pl.BlockSpec((_NC, bn, k), lambda i: (0, i, 0)),
            pl.BlockSpec((bn, din), lambda i: (i, 0)),
            full(din, hid), full(1, hid),
        ],
        out_specs=[pl.BlockSpec((bn, hid), lambda i: (i, 0)),
                   pl.BlockSpec((bn, 1), lambda i: (i, 0))],
        out_shape=[jax.ShapeDtypeStruct((n, hid), f32),
                   jax.ShapeDtypeStruct((n, 1), f32)],
    )(agg1, cnt, x, root1.T.astype(f32), b1[None, :].astype(f32))

    gathh = pl.kernel(
        functools.partial(_gather1_body, nu, tail),
        out_type=jax.ShapeDtypeStruct((e, hid), f32),
        mesh=_mesh(),
        compiler_params=_SC_PARAMS,
        scratch_types=(
            pltpu.VMEM((8, _GRP), jnp.int32),
            pltpu.VMEM((16 * _GRP, hid), f32),
            pltpu.SemaphoreType.DMA,
            pltpu.SemaphoreType.DMA,
        ),
    )
    hs = gathh(h, rowr).reshape(nrow16, 16 * hid)

    msg2 = pl.pallas_call(
        functools.partial(_edge2_body, k),
        grid=(nrow16 // br,),
        in_specs=[blk(br, 16 * hid), blk(br, 128), full(k, 16 * hid, 128),
                  full(k, 128, 16 * dout)],
        out_specs=blk(br, 128),
        out_shape=jax.ShapeDtypeStruct((nrow16, 128), f32),
    )(hs, gs2, g2k, ek2)

    zo = jnp.zeros((n, dout), f32)
    scat2 = pl.kernel(
        functools.partial(_scatter_body, nu, tail, False),
        out_type=jax.ShapeDtypeStruct((_NC, n, dout), f32),
        mesh=_mesh(),
        compiler_params=_SC_PARAMS,
        scratch_types=(
            pltpu.VMEM((16, _GRP), jnp.int32),
            pltpu.VMEM((16 * _GRP, dout), f32),
            pltpu.VMEM_SHARED((n, dout), f32),
            pltpu.SemaphoreType.DMA,
            pltpu.SemaphoreType.DMA,
        ),
    )
    agg2 = scat2(msg2.reshape(e, dout), colr, zo)

    out = pl.pallas_call(
        _node2_body,
        grid=(n // bn,),
        in_specs=[
            pl.BlockSpec((_NC, bn, dout), lambda i: (0, i, 0)),
            pl.BlockSpec((bn, 1), lambda i: (i, 0)),
            pl.BlockSpec((bn, hid), lambda i: (i, 0)),
            full(hid, dout), full(1, dout),
        ],
        out_specs=pl.BlockSpec((bn, dout), lambda i: (i, 0)),
        out_shape=jax.ShapeDtypeStruct((n, dout), f32),
    )(agg2, invc, h, root2.T.astype(f32), b2[None, :].astype(f32))
    return out
